# baseline (speedup 1.0000x reference)
import functools

import jax
import jax.numpy as jnp
from jax import lax
from jax.experimental import pallas as pl
from jax.experimental.pallas import tpu as pltpu
from jax.experimental.pallas import tpu_sc as plsc

NSC = 2
NSUB = 16
LANES = 16
NW = NSC * NSUB

BN_EPS = 1e-5
F32 = jnp.float32


def _bf16_round(x):
    u = lax.bitcast_convert_type(x, jnp.uint32)
    r = (u + jnp.uint32(0x7FFF) + ((u >> jnp.uint32(16)) & jnp.uint32(1))) \
        & jnp.uint32(0xFFFF0000)
    return lax.bitcast_convert_type(r, F32)


@functools.cache
def _make_sc_aggr(N, E, H, ED, C, RPW):
    HL = H // LANES
    TR = RPW
    ACC_R = RPW + 8
    NREM = N - (NW - 1) * RPW
    CB = C // LANES

    mesh = plsc.VectorSubcoreMesh(core_axis_name="c", subcore_axis_name="s")

    @functools.partial(
        pl.kernel,
        out_type=jax.ShapeDtypeStruct((N, H), F32),
        mesh=mesh,
        scratch_types=[
            pltpu.VMEM((ED, H), F32),
            pltpu.VMEM((H,), F32),
            pltpu.VMEM((LANES,), jnp.int32),
            pltpu.VMEM((LANES,), jnp.int32),
            pltpu.VMEM((C,), jnp.int32),
            pltpu.VMEM((C,), jnp.int32),
            pltpu.VMEM((C * ED + LANES,), F32),
            pltpu.VMEM((C, H), F32),
            pltpu.VMEM((ACC_R, H), F32),
            pltpu.SemaphoreType.DMA,
        ],
    )
    def sc_aggr(h_hbm, src_hbm, dst_hbm, ea_hbm, lo_hbm, hi_hbm, ew_hbm,
                eb_hbm, out_hbm, ew_v, eb_v, lo_v, hi_v, src_v, dst_v, ea_v,
                rows_v, acc_v, sem):
        cid = lax.axis_index("c")
        sid = lax.axis_index("s")
        wid = sid * NSC + cid
        row0 = wid * RPW

        pltpu.sync_copy(ew_hbm, ew_v)
        pltpu.sync_copy(eb_hbm, eb_v)
        woff = pl.multiple_of(wid * LANES, 8)
        pltpu.sync_copy(lo_hbm.at[pl.ds(woff, LANES)], lo_v)
        pltpu.sync_copy(hi_hbm.at[pl.ds(woff, LANES)], hi_v)

        zvec = jnp.zeros((LANES,), F32)

        def zrow(i, carry):
            for j in range(HL):
                acc_v[i, pl.ds(j * LANES, LANES)] = zvec
            return carry

        lax.fori_loop(0, ACC_R, zrow, 0)

        lo_w = lo_v[pl.ds(0, LANES)][0]
        hi_w = hi_v[pl.ds(0, LANES)][0]
        s0 = (lo_w // 8) * 8
        nch = (hi_w - s0 + C - 1) // C

        ebs = [eb_v[pl.ds(j * LANES, LANES)] for j in range(HL)]
        ews = [[ew_v[m, pl.ds(j * LANES, LANES)] for j in range(HL)]
               for m in range(ED)]

        def chunk(k, carry):
            off = pl.multiple_of(s0 + k * C, 8)
            pltpu.sync_copy(src_hbm.at[pl.ds(off, C)], src_v)
            pltpu.sync_copy(dst_hbm.at[pl.ds(off, C)], dst_v)
            pltpu.sync_copy(ea_hbm.at[pl.ds(off * ED, C * ED)],
                            ea_v.at[pl.ds(0, C * ED)])
            pltpu.async_copy(h_hbm.at[src_v], rows_v, sem).wait()

            def blk(b, c2):
                dvec = dst_v[pl.ds(b * LANES, LANES)]
                for m in range(LANES):
                    i = b * LANES + m
                    eg = off + i
                    valid = jnp.logical_and(eg >= lo_w, eg < hi_w)
                    dloc = jnp.where(valid, dvec[m] - row0, TR)
                    av = ea_v[pl.ds(i * ED, LANES)]
                    a = [av[mm] for mm in range(ED)]
                    for j in range(HL):
                        sl = pl.ds(j * LANES, LANES)
                        e = a[0] * ews[0][j]
                        for mm in range(1, ED):
                            e = e + a[mm] * ews[mm][j]
                        e = e + ebs[j]
                        msgv = jnp.maximum(rows_v[i, sl] + e, 0.0)
                        acc_v[dloc, sl] = acc_v[dloc, sl] + msgv
                return c2

            lax.fori_loop(0, CB, blk, 0)
            return carry

        lax.fori_loop(0, nch, chunk, 0)

        @pl.when(row0 + RPW <= N)
        def _():
            pltpu.sync_copy(
                acc_v.at[pl.ds(0, RPW), :],
                out_hbm.at[pl.ds(pl.multiple_of(row0, 8), RPW), :])

        @pl.when(jnp.logical_and(row0 + RPW > N, row0 < N))
        def _():
            pltpu.sync_copy(
                acc_v.at[pl.ds(0, NREM), :],
                out_hbm.at[pl.ds(pl.multiple_of(row0, 8), NREM), :])

    return sc_aggr


def _encoder_body(x_ref, w_ref, b_ref, o_ref):
    o_ref[...] = (
        jnp.dot(x_ref[...], w_ref[...], preferred_element_type=F32) + b_ref[...]
    )


def _layer_body(h_ref, a0_ref, w1_ref, b1_ref, w2_ref, b2_ref,
                g_ref, be_ref, o_ref):
    z = h_ref[...] + a0_ref[...]
    t = jnp.dot(z, w1_ref[...], preferred_element_type=F32) + b1_ref[...]
    t = jnp.maximum(t, 0.0)
    z2 = jnp.dot(t, w2_ref[...], preferred_element_type=F32) + b2_ref[...]
    n = z2.shape[0]
    mu = jnp.sum(z2, axis=0, keepdims=True) * (1.0 / n)
    d = z2 - mu
    var = jnp.sum(d * d, axis=0, keepdims=True) * (1.0 / n)
    zn = d * lax.rsqrt(var + BN_EPS) * g_ref[...] + be_ref[...]
    o_ref[...] = jnp.maximum(zn, 0.0)


def _readout_body(h_ref, m_ref, w1_ref, b1_ref, pa_ref, w2_ref, b2_ref, o_ref):
    sel = h_ref[...] * m_ref[...]
    y = jnp.dot(sel, w1_ref[...], preferred_element_type=F32) + b1_ref[...]
    y = jnp.where(y >= 0.0, y, pa_ref[0, 0] * y)
    o_ref[...] = (
        jnp.dot(y, w2_ref[...], preferred_element_type=F32) + b2_ref[...]
    )


def _tc_call(body, out_shape, *args):
    return pl.pallas_call(body, out_shape=out_shape)(*args)


def kernel(x, edge_attr, edge_index, mask, nW, nb, eW, eb, W1, b1, W2, b2,
           g, beta, rW1, rb1, pa, rW2, rb2):
    N, D = x.shape
    E, ED = edge_attr.shape
    H = nW.shape[1]
    L = eW.shape[0]

    C = 80
    RPW = (((N + NW - 1) // NW) + 7) // 8 * 8
    src = edge_index[0]
    dst = edge_index[1]
    perm = jnp.argsort(dst, stable=True).astype(jnp.int32)
    dst_s = jnp.take(dst, perm)
    src_s = jnp.take(src, perm)
    ea_s = jnp.take(lax.optimization_barrier(_bf16_round(edge_attr)),
                    perm, axis=0)
    lo = jnp.searchsorted(
        dst_s, jnp.arange(NW, dtype=jnp.int32) * RPW).astype(jnp.int32)
    hi = jnp.concatenate([lo[1:], jnp.array([E], jnp.int32)])
    lo = jnp.repeat(lo, LANES)
    hi = jnp.repeat(hi, LANES)
    src_p = jnp.concatenate([src_s, jnp.zeros((C,), jnp.int32)])
    dst_p = jnp.concatenate([dst_s, jnp.zeros((C,), jnp.int32)])
    ea_p = jnp.concatenate([ea_s, jnp.zeros((C, ED), F32)]).reshape(-1)

    eW_r = lax.optimization_barrier(_bf16_round(eW))
    sc_aggr = _make_sc_aggr(N, E, H, ED, C, RPW)

    h = _tc_call(_encoder_body, jax.ShapeDtypeStruct((N, H), F32),
                 x, nW, nb.reshape(1, H))

    for l in range(L):
        aggr = sc_aggr(h, src_p, dst_p, ea_p, lo, hi, eW_r[l], eb[l])
        h = _tc_call(
            _layer_body, jax.ShapeDtypeStruct((N, H), F32),
            h, aggr, W1[l], b1[l].reshape(1, -1),
            W2[l], b2[l].reshape(1, H), g[l].reshape(1, H),
            beta[l].reshape(1, H))

    maskf = mask.astype(F32).reshape(N, 1)
    out = _tc_call(
        _readout_body, jax.ShapeDtypeStruct((N, 1), F32),
        h, maskf, rW1, rb1.reshape(1, -1), pa.reshape(1, 1),
        rW2, rb2.reshape(1, 1))
    return out[:, 0]


# speedup vs baseline: 1.0824x; 1.0824x over previous
import functools

import jax
import jax.numpy as jnp
from jax import lax
from jax.experimental import pallas as pl
from jax.experimental.pallas import tpu as pltpu
from jax.experimental.pallas import tpu_sc as plsc

NSC = 2
NSUB = 16
LANES = 16
NW = NSC * NSUB

BN_EPS = 1e-5
F32 = jnp.float32


def _bf16_round(x):
    u = lax.bitcast_convert_type(x, jnp.uint32)
    r = (u + jnp.uint32(0x7FFF) + ((u >> jnp.uint32(16)) & jnp.uint32(1))) \
        & jnp.uint32(0xFFFF0000)
    return lax.bitcast_convert_type(r, F32)


@functools.cache
def _make_sc_aggr(N, E, H, ED, C, RPW):
    HL = H // LANES
    TR = RPW
    ACC_R = RPW + 8
    NREM = N - (NW - 1) * RPW
    CB = C // LANES

    mesh = plsc.VectorSubcoreMesh(core_axis_name="c", subcore_axis_name="s")

    @functools.partial(
        pl.kernel,
        out_type=jax.ShapeDtypeStruct((N, H), F32),
        mesh=mesh,
        scratch_types=[
            pltpu.VMEM((ED, H), F32),
            pltpu.VMEM((H,), F32),
            pltpu.VMEM((LANES,), jnp.int32),
            pltpu.VMEM((LANES,), jnp.int32),
            pltpu.VMEM((C,), jnp.int32),
            pltpu.VMEM((C,), jnp.int32),
            pltpu.VMEM((C * ED + LANES,), F32),
            pltpu.VMEM((C, H), F32),
            pltpu.VMEM((ACC_R, H), F32),
            pltpu.SemaphoreType.DMA,
        ],
    )
    def sc_aggr(h_hbm, src_hbm, dst_hbm, ea_hbm, lo_hbm, hi_hbm, ew_hbm,
                eb_hbm, out_hbm, ew_v, eb_v, lo_v, hi_v, src_v, dst_v, ea_v,
                rows_v, acc_v, sem):
        cid = lax.axis_index("c")
        sid = lax.axis_index("s")
        wid = sid * NSC + cid
        row0 = wid * RPW

        pltpu.sync_copy(ew_hbm, ew_v)
        pltpu.sync_copy(eb_hbm, eb_v)
        woff = pl.multiple_of(wid * LANES, 8)
        pltpu.sync_copy(lo_hbm.at[pl.ds(woff, LANES)], lo_v)
        pltpu.sync_copy(hi_hbm.at[pl.ds(woff, LANES)], hi_v)

        zvec = jnp.zeros((LANES,), F32)

        def zrow(i, carry):
            for j in range(HL):
                acc_v[i, pl.ds(j * LANES, LANES)] = zvec
            return carry

        lax.fori_loop(0, ACC_R, zrow, 0)

        lo_w = lo_v[pl.ds(0, LANES)][0]
        hi_w = hi_v[pl.ds(0, LANES)][0]
        s0 = (lo_w // 8) * 8
        nch = (hi_w - s0 + C - 1) // C

        ebs = [eb_v[pl.ds(j * LANES, LANES)] for j in range(HL)]
        ews = [[ew_v[m, pl.ds(j * LANES, LANES)] for j in range(HL)]
               for m in range(ED)]

        def chunk(k, carry):
            off = pl.multiple_of(s0 + k * C, 8)
            pltpu.sync_copy(src_hbm.at[pl.ds(off, C)], src_v)
            pltpu.sync_copy(dst_hbm.at[pl.ds(off, C)], dst_v)
            pltpu.sync_copy(ea_hbm.at[pl.ds(off * ED, C * ED)],
                            ea_v.at[pl.ds(0, C * ED)])
            pltpu.async_copy(h_hbm.at[src_v], rows_v, sem).wait()

            def blk(b, c2):
                dvec = dst_v[pl.ds(b * LANES, LANES)]
                for m in range(LANES):
                    i = b * LANES + m
                    eg = off + i
                    valid = jnp.logical_and(eg >= lo_w, eg < hi_w)
                    dloc = jnp.where(valid, dvec[m] - row0, TR)
                    av = ea_v[pl.ds(i * ED, LANES)]
                    a = [av[mm] for mm in range(ED)]
                    for j in range(HL):
                        sl = pl.ds(j * LANES, LANES)
                        e = a[0] * ews[0][j]
                        for mm in range(1, ED):
                            e = e + a[mm] * ews[mm][j]
                        e = e + ebs[j]
                        msgv = jnp.maximum(rows_v[i, sl] + e, 0.0)
                        acc_v[dloc, sl] = acc_v[dloc, sl] + msgv
                return c2

            lax.fori_loop(0, CB, blk, 0)
            return carry

        lax.fori_loop(0, nch, chunk, 0)

        @pl.when(row0 + RPW <= N)
        def _():
            pltpu.sync_copy(
                acc_v.at[pl.ds(0, RPW), :],
                out_hbm.at[pl.ds(pl.multiple_of(row0, 8), RPW), :])

        @pl.when(jnp.logical_and(row0 + RPW > N, row0 < N))
        def _():
            pltpu.sync_copy(
                acc_v.at[pl.ds(0, NREM), :],
                out_hbm.at[pl.ds(pl.multiple_of(row0, 8), NREM), :])

    return sc_aggr


def _encoder_body(x_ref, w_ref, b_ref, o_ref):
    o_ref[...] = (
        jnp.dot(x_ref[...], w_ref[...], preferred_element_type=F32) + b_ref[...]
    )


def _layer_body(h_ref, a0_ref, w1_ref, b1_ref, w2_ref, b2_ref,
                g_ref, be_ref, o_ref):
    z = h_ref[...] + a0_ref[...]
    t = jnp.dot(z, w1_ref[...], preferred_element_type=F32) + b1_ref[...]
    t = jnp.maximum(t, 0.0)
    z2 = jnp.dot(t, w2_ref[...], preferred_element_type=F32) + b2_ref[...]
    n = z2.shape[0]
    mu = jnp.sum(z2, axis=0, keepdims=True) * (1.0 / n)
    d = z2 - mu
    var = jnp.sum(d * d, axis=0, keepdims=True) * (1.0 / n)
    zn = d * lax.rsqrt(var + BN_EPS) * g_ref[...] + be_ref[...]
    o_ref[...] = jnp.maximum(zn, 0.0)


def _readout_body(h_ref, m_ref, w1_ref, b1_ref, pa_ref, w2_ref, b2_ref, o_ref):
    sel = h_ref[...] * m_ref[...]
    y = jnp.dot(sel, w1_ref[...], preferred_element_type=F32) + b1_ref[...]
    y = jnp.where(y >= 0.0, y, pa_ref[0, 0] * y)
    o_ref[...] = (
        jnp.dot(y, w2_ref[...], preferred_element_type=F32) + b2_ref[...]
    )


def _tc_call(body, out_shape, *args):
    return pl.pallas_call(body, out_shape=out_shape)(*args)


def kernel(x, edge_attr, edge_index, mask, nW, nb, eW, eb, W1, b1, W2, b2,
           g, beta, rW1, rb1, pa, rW2, rb2):
    N, D = x.shape
    E, ED = edge_attr.shape
    H = nW.shape[1]
    L = eW.shape[0]

    C = 128
    RPW = (((N + NW - 1) // NW) + 7) // 8 * 8
    src = edge_index[0]
    dst = edge_index[1]
    perm = jnp.argsort(dst, stable=True).astype(jnp.int32)
    dst_s = jnp.take(dst, perm)
    src_s = jnp.take(src, perm)
    ea_s = jnp.take(lax.optimization_barrier(_bf16_round(edge_attr)),
                    perm, axis=0)
    lo = jnp.searchsorted(
        dst_s, jnp.arange(NW, dtype=jnp.int32) * RPW).astype(jnp.int32)
    hi = jnp.concatenate([lo[1:], jnp.array([E], jnp.int32)])
    lo = jnp.repeat(lo, LANES)
    hi = jnp.repeat(hi, LANES)
    src_p = jnp.concatenate([src_s, jnp.zeros((C,), jnp.int32)])
    dst_p = jnp.concatenate([dst_s, jnp.zeros((C,), jnp.int32)])
    ea_p = jnp.concatenate([ea_s, jnp.zeros((C, ED), F32)]).reshape(-1)

    eW_r = lax.optimization_barrier(_bf16_round(eW))
    sc_aggr = _make_sc_aggr(N, E, H, ED, C, RPW)

    h = _tc_call(_encoder_body, jax.ShapeDtypeStruct((N, H), F32),
                 x, nW, nb.reshape(1, H))

    for l in range(L):
        aggr = sc_aggr(h, src_p, dst_p, ea_p, lo, hi, eW_r[l], eb[l])
        h = _tc_call(
            _layer_body, jax.ShapeDtypeStruct((N, H), F32),
            h, aggr, W1[l], b1[l].reshape(1, -1),
            W2[l], b2[l].reshape(1, H), g[l].reshape(1, H),
            beta[l].reshape(1, H))

    maskf = mask.astype(F32).reshape(N, 1)
    out = _tc_call(
        _readout_body, jax.ShapeDtypeStruct((N, 1), F32),
        h, maskf, rW1, rb1.reshape(1, -1), pa.reshape(1, 1),
        rW2, rb2.reshape(1, 1))
    return out[:, 0]
